# baseline (speedup 1.0000x reference)
import functools

import numpy as np
import jax
import jax.numpy as jnp
from jax import lax
from jax.experimental import pallas as pl
from jax.experimental.pallas import tpu as pltpu
from jax.experimental.pallas import tpu_sc as plsc

N_EMBD = 1024
N_HEAD = 16
HEAD_DIM = 64
KV_LORA = 256
ROPE_DIM = 32
NOPE_DIM = HEAD_DIM - ROPE_DIM
N_EXP = 8
TOP_K = 2
INTER = 2048
THETA = 100000.0

BT1 = 256
BTQ = 512
BTE = 256
BTS = 256

HALF = ROPE_DIM // 2


def _rope_tables(T):
    freqs = 1.0 / (THETA ** (np.arange(0, ROPE_DIM, 2, dtype=np.float32) / ROPE_DIM))
    t = np.arange(T, dtype=np.float32)
    f = np.outer(t, freqs)
    return np.cos(f).astype(np.float32), np.sin(f).astype(np.float32)


def _deinterleave_perm():
    p = np.empty((ROPE_DIM,), dtype=np.int32)
    p[:HALF] = 2 * np.arange(HALF)
    p[HALF:] = 2 * np.arange(HALF) + 1
    return p


def _weight_perms():
    pr = _deinterleave_perm()
    qperm = np.arange(N_HEAD * HEAD_DIM, dtype=np.int32)
    for h in range(N_HEAD):
        base = h * HEAD_DIM + NOPE_DIM
        qperm[base:base + ROPE_DIM] = base + pr
    kvaperm = np.arange(KV_LORA + ROPE_DIM, dtype=np.int32)
    kvaperm[KV_LORA:] = KV_LORA + pr
    return qperm, kvaperm


def _k1_body(x_ref, ln1_ref, wq_ref, wkva_ref, wkvb_ref, cos_ref, sin_ref,
             q_ref, knope_ref, krope_ref):
    x = x_ref[...]
    xn = x * lax.rsqrt(jnp.mean(x * x, axis=1, keepdims=True) + 1e-6) * ln1_ref[...]
    xnb = xn.astype(jnp.bfloat16)
    q = jnp.dot(xnb, wq_ref[...].astype(jnp.bfloat16),
                preferred_element_type=jnp.float32)
    ckv = jnp.dot(xnb, wkva_ref[...].astype(jnp.bfloat16),
                  preferred_element_type=jnp.float32)
    latent = ckv[:, :KV_LORA]
    kr = ckv[:, KV_LORA:]
    knope_ref[...] = jnp.dot(latent.astype(jnp.bfloat16),
                             wkvb_ref[...].astype(jnp.bfloat16),
                             preferred_element_type=jnp.float32)
    cos = cos_ref[...]
    sin = sin_ref[...]
    q_ref[...] = q
    for h in range(N_HEAD):
        base = h * HEAD_DIM + NOPE_DIM
        a = q[:, base:base + HALF]
        b = q[:, base + HALF:base + ROPE_DIM]
        q_ref[:, base:base + HALF] = a * cos - b * sin
        q_ref[:, base + HALF:base + ROPE_DIM] = a * sin + b * cos
    a = kr[:, :HALF]
    b = kr[:, HALF:]
    krope_ref[...] = jnp.concatenate([a * cos - b * sin, a * sin + b * cos], axis=1)


def _proj(xf, ln1_w, Wq_p, Wkva_p, Wkvb, cosT, sinT):
    T = xf.shape[0]
    nb = T // BT1
    return pl.pallas_call(
        _k1_body,
        grid=(nb,),
        in_specs=[
            pl.BlockSpec((BT1, N_EMBD), lambda i: (i, 0)),
            pl.BlockSpec((1, N_EMBD), lambda i: (0, 0)),
            pl.BlockSpec((N_EMBD, N_HEAD * HEAD_DIM), lambda i: (0, 0)),
            pl.BlockSpec((N_EMBD, KV_LORA + ROPE_DIM), lambda i: (0, 0)),
            pl.BlockSpec((KV_LORA, N_HEAD * NOPE_DIM), lambda i: (0, 0)),
            pl.BlockSpec((BT1, HALF), lambda i: (i, 0)),
            pl.BlockSpec((BT1, HALF), lambda i: (i, 0)),
        ],
        out_specs=[
            pl.BlockSpec((BT1, N_HEAD * HEAD_DIM), lambda i: (i, 0)),
            pl.BlockSpec((BT1, N_HEAD * NOPE_DIM), lambda i: (i, 0)),
            pl.BlockSpec((BT1, ROPE_DIM), lambda i: (i, 0)),
        ],
        out_shape=[
            jax.ShapeDtypeStruct((T, N_HEAD * HEAD_DIM), jnp.float32),
            jax.ShapeDtypeStruct((T, N_HEAD * NOPE_DIM), jnp.float32),
            jax.ShapeDtypeStruct((T, ROPE_DIM), jnp.float32),
        ],
    )(xf, ln1_w, Wq_p, Wkva_p, Wkvb, cosT, sinT)


def _att_body_iq(iq, q_ref, kT_ref, k_ref, y_ref):
    L = (iq + 1) * BTQ
    q = q_ref[0].astype(jnp.bfloat16)
    kT = kT_ref[0].astype(jnp.bfloat16)
    v = k_ref[0].astype(jnp.bfloat16)
    s_diag = jnp.dot(q, kT[:, L - BTQ:],
                     preferred_element_type=jnp.float32) * 0.125
    row = lax.broadcasted_iota(jnp.int32, (BTQ, BTQ), 0)
    col = lax.broadcasted_iota(jnp.int32, (BTQ, BTQ), 1)
    s_diag = jnp.where(col <= row, s_diag, jnp.float32(-1e9))
    if iq == 0:
        m = jnp.max(s_diag, axis=1, keepdims=True)
        p = jnp.exp(s_diag - m).astype(jnp.bfloat16)
        l = jnp.sum(p.astype(jnp.float32), axis=1, keepdims=True)
        y = jnp.dot(p, v, preferred_element_type=jnp.float32)
    else:
        s_pre = jnp.dot(q, kT[:, :L - BTQ],
                        preferred_element_type=jnp.float32) * 0.125
        m = jnp.maximum(jnp.max(s_pre, axis=1, keepdims=True),
                        jnp.max(s_diag, axis=1, keepdims=True))
        p_pre = jnp.exp(s_pre - m).astype(jnp.bfloat16)
        p_diag = jnp.exp(s_diag - m).astype(jnp.bfloat16)
        l = (jnp.sum(p_pre.astype(jnp.float32), axis=1, keepdims=True)
             + jnp.sum(p_diag.astype(jnp.float32), axis=1, keepdims=True))
        y = (jnp.dot(p_pre, v[:L - BTQ], preferred_element_type=jnp.float32)
             + jnp.dot(p_diag, v[L - BTQ:], preferred_element_type=jnp.float32))
    y_ref[0] = y / l


def _attention(q3, kT3, k3):
    T = q3.shape[1]
    nq = T // BTQ
    pieces = []
    for iq in range(nq):
        L = (iq + 1) * BTQ
        pieces.append(pl.pallas_call(
            functools.partial(_att_body_iq, iq),
            grid=(N_HEAD,),
            in_specs=[
                pl.BlockSpec((1, BTQ, HEAD_DIM), lambda h, iq=iq: (h, iq, 0)),
                pl.BlockSpec((1, HEAD_DIM, L), lambda h: (h, 0, 0)),
                pl.BlockSpec((1, L, HEAD_DIM), lambda h: (h, 0, 0)),
            ],
            out_specs=pl.BlockSpec((1, BTQ, HEAD_DIM), lambda h: (h, 0, 0)),
            out_shape=jax.ShapeDtypeStruct((N_HEAD, BTQ, HEAD_DIM), jnp.float32),
        )(q3, kT3, k3))
    return jnp.concatenate(pieces, axis=1)


def _oproj_gate_body(x_ref, y_ref, wo_ref, ln2_ref, gw_ref, bias_ref,
                     h_ref, xn2_ref, ti_ref, tw_ref, rk_ref, cnt_ref,
                     carry_ref):
    h = x_ref[...] + jnp.dot(y_ref[...].astype(jnp.bfloat16),
                             wo_ref[...].astype(jnp.bfloat16),
                             preferred_element_type=jnp.float32)
    h_ref[...] = h
    xn = h * lax.rsqrt(jnp.mean(h * h, axis=1, keepdims=True) + 1e-6) * ln2_ref[...]
    xn2_ref[...] = xn
    logits = jnp.dot(xn, gw_ref[...], preferred_element_type=jnp.float32) + bias_ref[...]
    m = jnp.max(logits, axis=1, keepdims=True)
    e = jnp.exp(logits - m)
    probs = e / jnp.sum(e, axis=1, keepdims=True)
    lane = lax.broadcasted_iota(jnp.int32, probs.shape, 1)
    m1 = jnp.max(probs, axis=1, keepdims=True)
    sel1 = jnp.min(jnp.where(probs == m1, lane, 99), axis=1, keepdims=True)
    p2 = jnp.where(lane == sel1, jnp.float32(-1.0), probs)
    m2 = jnp.max(p2, axis=1, keepdims=True)
    sel2 = jnp.min(jnp.where(p2 == m2, lane, 99), axis=1, keepdims=True)
    denom = m1 + m2
    ti_ref[...] = jnp.where(lane == 0, sel1, jnp.where(lane == 1, sel2, 0))
    tw_ref[...] = jnp.where(lane == 0, m1 / denom,
                            jnp.where(lane == 1, m2 / denom, jnp.float32(0.0)))

    @pl.when(pl.program_id(0) == 0)
    def _():
        carry_ref[...] = jnp.zeros_like(carry_ref)

    BT = probs.shape[0]
    ohA = (lane == sel1).astype(jnp.float32)
    ohB = (lane == sel2).astype(jnp.float32)
    tot = ohA + ohB
    r_i = lax.broadcasted_iota(jnp.int32, (BT, BT), 0)
    c_i = lax.broadcasted_iota(jnp.int32, (BT, BT), 1)
    tril_s = (c_i < r_i).astype(jnp.float32)
    prefix = jnp.dot(tril_s, tot, preferred_element_type=jnp.float32)
    base = carry_ref[...] + prefix
    rkA = jnp.sum(ohA * base, axis=1, keepdims=True)
    rkB = jnp.sum(ohB * (base + ohA), axis=1, keepdims=True)
    rk_ref[...] = jnp.where(lane == 0, rkA.astype(jnp.int32),
                            jnp.where(lane == 1, rkB.astype(jnp.int32), 0))
    new_carry = carry_ref[...] + jnp.sum(tot, axis=0, keepdims=True)
    carry_ref[...] = new_carry
    cnt_ref[...] = new_carry.astype(jnp.int32)


def _oproj_gate(xf, y, Wo_p, ln2_w, gateW, bias):
    T = xf.shape[0]
    return pl.pallas_call(
        _oproj_gate_body,
        grid=(T // BT1,),
        in_specs=[
            pl.BlockSpec((BT1, N_EMBD), lambda i: (i, 0)),
            pl.BlockSpec((BT1, N_HEAD * HEAD_DIM), lambda i: (i, 0)),
            pl.BlockSpec((N_HEAD * HEAD_DIM, N_EMBD), lambda i: (0, 0)),
            pl.BlockSpec((1, N_EMBD), lambda i: (0, 0)),
            pl.BlockSpec((N_EMBD, N_EXP), lambda i: (0, 0)),
            pl.BlockSpec((1, N_EXP), lambda i: (0, 0)),
        ],
        out_specs=[
            pl.BlockSpec((BT1, N_EMBD), lambda i: (i, 0)),
            pl.BlockSpec((BT1, N_EMBD), lambda i: (i, 0)),
            pl.BlockSpec((BT1, N_EXP), lambda i: (i, 0)),
            pl.BlockSpec((BT1, N_EXP), lambda i: (i, 0)),
            pl.BlockSpec((BT1, N_EXP), lambda i: (i, 0)),
            pl.BlockSpec((1, N_EXP), lambda i: (0, 0)),
        ],
        out_shape=[
            jax.ShapeDtypeStruct((T, N_EMBD), jnp.float32),
            jax.ShapeDtypeStruct((T, N_EMBD), jnp.float32),
            jax.ShapeDtypeStruct((T, N_EXP), jnp.int32),
            jax.ShapeDtypeStruct((T, N_EXP), jnp.float32),
            jax.ShapeDtypeStruct((T, N_EXP), jnp.int32),
            jax.ShapeDtypeStruct((1, N_EXP), jnp.int32),
        ],
        scratch_shapes=[pltpu.VMEM((1, N_EXP), jnp.float32)],
    )(xf, y, Wo_p, ln2_w, gateW, bias)


def _expert_body(be_ref, act_ref, xs_ref, sw_ref, gw_ref, uw_ref, dw_ref, ys_ref):
    i = pl.program_id(0)

    @pl.when(act_ref[i] != 0)
    def _():
        x = xs_ref[...].astype(jnp.bfloat16)
        g = jnp.dot(x, gw_ref[0].astype(jnp.bfloat16),
                    preferred_element_type=jnp.float32)
        u = jnp.dot(x, uw_ref[0].astype(jnp.bfloat16),
                    preferred_element_type=jnp.float32)
        a = (g * jax.nn.sigmoid(g) * u).astype(jnp.bfloat16)
        ys_ref[...] = jnp.dot(a, dw_ref[0].astype(jnp.bfloat16),
                              preferred_element_type=jnp.float32) * sw_ref[...]


def _experts(be, act, xs, swt, gw, uw, dw):
    NBT = xs.shape[0]
    NB = NBT // BTE
    grid_spec = pltpu.PrefetchScalarGridSpec(
        num_scalar_prefetch=2,
        grid=(NB,),
        in_specs=[
            pl.BlockSpec((BTE, N_EMBD), lambda i, be, act: (i, 0)),
            pl.BlockSpec((BTE, 1), lambda i, be, act: (i, 0)),
            pl.BlockSpec((1, N_EMBD, INTER), lambda i, be, act: (be[i], 0, 0)),
            pl.BlockSpec((1, N_EMBD, INTER), lambda i, be, act: (be[i], 0, 0)),
            pl.BlockSpec((1, INTER, N_EMBD), lambda i, be, act: (be[i], 0, 0)),
        ],
        out_specs=pl.BlockSpec((BTE, N_EMBD), lambda i, be, act: (i, 0)),
    )
    return pl.pallas_call(
        _expert_body,
        grid_spec=grid_spec,
        out_shape=jax.ShapeDtypeStruct((NBT, N_EMBD), jnp.float32),
    )(be, act, xs, swt, gw, uw, dw)


def _shared_body(h_ref, xn2_ref, sgw_ref, suw_ref, sdw_ref, hs_ref):
    x = xn2_ref[...].astype(jnp.bfloat16)
    g = jnp.dot(x, sgw_ref[...].astype(jnp.bfloat16),
                preferred_element_type=jnp.float32)
    u = jnp.dot(x, suw_ref[...].astype(jnp.bfloat16),
                preferred_element_type=jnp.float32)
    a = (g * jax.nn.sigmoid(g) * u).astype(jnp.bfloat16)
    hs_ref[...] = h_ref[...] + jnp.dot(a, sdw_ref[...].astype(jnp.bfloat16),
                                       preferred_element_type=jnp.float32)


def _shared_ffn(h, xn2, sgw, suw, sdw):
    T = h.shape[0]
    return pl.pallas_call(
        _shared_body,
        grid=(T // BTS,),
        in_specs=[
            pl.BlockSpec((BTS, N_EMBD), lambda i: (i, 0)),
            pl.BlockSpec((BTS, N_EMBD), lambda i: (i, 0)),
            pl.BlockSpec((N_EMBD, INTER), lambda i: (0, 0)),
            pl.BlockSpec((N_EMBD, INTER), lambda i: (0, 0)),
            pl.BlockSpec((INTER, N_EMBD), lambda i: (0, 0)),
        ],
        out_specs=pl.BlockSpec((BTS, N_EMBD), lambda i: (i, 0)),
        out_shape=jax.ShapeDtypeStruct((T, N_EMBD), jnp.float32),
    )(h, xn2, sgw, suw, sdw)


def _final_body(hs_ref, y0_ref, y1_ref, out_ref):
    out_ref[...] = hs_ref[...] + y0_ref[...] + y1_ref[...]


def _combine(hs, yall):
    T = hs.shape[0]
    nb = T // BTS
    return pl.pallas_call(
        _final_body,
        grid=(nb,),
        in_specs=[
            pl.BlockSpec((BTS, N_EMBD), lambda i: (i, 0)),
            pl.BlockSpec((BTS, N_EMBD), lambda i: (i, 0)),
            pl.BlockSpec((BTS, N_EMBD), lambda i, nb=nb: (i + nb, 0)),
        ],
        out_specs=pl.BlockSpec((BTS, N_EMBD), lambda i: (i, 0)),
        out_shape=jax.ShapeDtypeStruct((T, N_EMBD), jnp.float32),
    )(hs, yall, yall)


def _sc_scatter_stream(pos, pw, zeros_i, zeros_f):
    P2 = pos.shape[0]
    NBT = zeros_i.shape[0]
    mesh = plsc.VectorSubcoreMesh(core_axis_name="c", subcore_axis_name="s")

    @functools.partial(
        pl.kernel, mesh=mesh,
        out_type=[jax.ShapeDtypeStruct((NBT,), jnp.int32),
                  jax.ShapeDtypeStruct((NBT,), jnp.float32)],
        compiler_params=pltpu.CompilerParams(needs_layout_passes=False),
        scratch_types=[pltpu.VMEM((P2,), jnp.int32),
                       pltpu.VMEM((P2,), jnp.float32),
                       pltpu.VMEM((NBT,), jnp.int32),
                       pltpu.VMEM((NBT,), jnp.float32)],
    )
    def k(pos_hbm, pw_hbm, zi_hbm, zf_hbm, sti_hbm, swt_hbm,
          pos_v, pw_v, sti_v, swt_v):
        wid = lax.axis_index("s") * 2 + lax.axis_index("c")

        @pl.when(wid == 0)
        def _():
            pltpu.sync_copy(pos_hbm, pos_v)
            pltpu.sync_copy(pw_hbm, pw_v)
            pltpu.sync_copy(zi_hbm, sti_v)
            pltpu.sync_copy(zf_hbm, swt_v)

            @pl.loop(0, P2 // 16)
            def body(i):
                idx = pos_v[pl.ds(i * 16, 16)]
                toks = (i * 16
                        + lax.broadcasted_iota(jnp.int32, (16,), 0)) // TOP_K
                plsc.store_scatter(sti_v, [idx], toks)
                plsc.store_scatter(swt_v, [idx], pw_v[pl.ds(i * 16, 16)])

            pltpu.sync_copy(sti_v, sti_hbm)
            pltpu.sync_copy(swt_v, swt_hbm)

    return k(pos, pw, zeros_i, zeros_f)


def _sc_gather_rows(table, idx, chunk):
    V, D = table.shape
    B = idx.shape[0]
    NW = 32
    b_per_w = B // NW
    nch = b_per_w // chunk
    mesh = plsc.VectorSubcoreMesh(core_axis_name="c", subcore_axis_name="s")

    @functools.partial(
        pl.kernel, mesh=mesh,
        out_type=jax.ShapeDtypeStruct((B, D), jnp.float32),
        scratch_types=[pltpu.VMEM((chunk,), jnp.int32),
                       pltpu.VMEM((chunk, D), jnp.float32),
                       pltpu.SemaphoreType.DMA],
    )
    def k(table_hbm, idx_hbm, out_hbm, idx_v, rows_v, sem):
        wid = lax.axis_index("s") * 2 + lax.axis_index("c")
        base = wid * b_per_w
        for c in range(nch):
            off = base + c * chunk
            pltpu.sync_copy(idx_hbm.at[pl.ds(off, chunk)], idx_v)
            pltpu.async_copy(table_hbm.at[idx_v], rows_v, sem).wait()
            pltpu.sync_copy(rows_v, out_hbm.at[pl.ds(off, chunk)])

    return k(table, idx)


def _route_meta(ti, tw, rk, cnt, T):
    P2 = T * TOP_K
    eid = ti[:, :TOP_K].reshape(-1)
    pw = tw[:, :TOP_K].reshape(-1)
    rank_within = rk[:, :TOP_K].reshape(-1)
    counts = cnt[0]
    nblk = (counts + BTE - 1) // BTE
    ends = jnp.cumsum(nblk)
    bstart = ends - nblk
    pos = jnp.take(bstart, eid) * BTE + rank_within
    NB = P2 // BTE + N_EXP
    bids = jnp.arange(NB, dtype=jnp.int32)
    be = jnp.searchsorted(ends, bids, side='right').astype(jnp.int32)
    act = (bids < ends[-1]).astype(jnp.int32)
    be = jnp.where(act == 1, be, 0)
    return pos, pw, be, act


def kernel(x, ln1_w, ln2_w, Wq, Wkva, Wkvb, Wo, gateW, expert_bias, gw, uw, dw, sgw, suw, sdw):
    B, T, C = x.shape
    xf = x.reshape(T, C)

    cos_np, sin_np = _rope_tables(T)
    cosT = jnp.asarray(cos_np)
    sinT = jnp.asarray(sin_np)
    qperm, kvaperm = _weight_perms()
    Wq_p = jnp.take(Wq, qperm, axis=1)
    Wkva_p = jnp.take(Wkva, kvaperm, axis=1)
    Wo_p = jnp.take(Wo, qperm, axis=0)

    q, knope, krope = _proj(xf, ln1_w.reshape(1, C), Wq_p, Wkva_p, Wkvb, cosT, sinT)
    q3 = q.reshape(T, N_HEAD, HEAD_DIM).transpose(1, 0, 2)
    knope3 = knope.reshape(T, N_HEAD, NOPE_DIM).transpose(1, 0, 2)
    k3 = jnp.concatenate(
        [knope3, jnp.broadcast_to(krope[None], (N_HEAD, T, ROPE_DIM))], axis=2)
    kT3 = k3.transpose(0, 2, 1)
    y3 = _attention(q3, kT3, k3)
    y = y3.transpose(1, 0, 2).reshape(T, N_HEAD * HEAD_DIM)
    h, xn2, ti, tw, rk, cnt = _oproj_gate(xf, y, Wo_p, ln2_w.reshape(1, C),
                                          gateW, expert_bias.reshape(1, N_EXP))
    pos, pw, be, act = _route_meta(ti, tw, rk, cnt, T)
    NBT = (T * TOP_K // BTE + N_EXP) * BTE

    hs = _shared_ffn(h, xn2, sgw, suw, sdw)
    sti, swt = _sc_scatter_stream(pos, pw, jnp.zeros((NBT,), jnp.int32),
                                  jnp.zeros((NBT,), jnp.float32))
    xs = jnp.take(xn2, sti, axis=0)
    ysw = _experts(be, act, xs, swt[:, None], gw, uw, dw)
    posT = pos.reshape(T, TOP_K)
    pos2 = jnp.concatenate([posT[:, 0], posT[:, 1]])
    yall = jnp.take(ysw, pos2, axis=0)

    out = _combine(hs, yall)
    return out.reshape(B, T, C)


# speedup vs baseline: 1.0204x; 1.0204x over previous
import functools

import numpy as np
import jax
import jax.numpy as jnp
from jax import lax
from jax.experimental import pallas as pl
from jax.experimental.pallas import tpu as pltpu
from jax.experimental.pallas import tpu_sc as plsc

N_EMBD = 1024
N_HEAD = 16
HEAD_DIM = 64
KV_LORA = 256
ROPE_DIM = 32
NOPE_DIM = HEAD_DIM - ROPE_DIM
N_EXP = 8
TOP_K = 2
INTER = 2048
THETA = 100000.0

BT1 = 256
BTQ = 512
BTE = 256
BTS = 256

HALF = ROPE_DIM // 2


def _rope_tables(T):
    freqs = 1.0 / (THETA ** (np.arange(0, ROPE_DIM, 2, dtype=np.float32) / ROPE_DIM))
    t = np.arange(T, dtype=np.float32)
    f = np.outer(t, freqs)
    return np.cos(f).astype(np.float32), np.sin(f).astype(np.float32)


def _deinterleave_perm():
    p = np.empty((ROPE_DIM,), dtype=np.int32)
    p[:HALF] = 2 * np.arange(HALF)
    p[HALF:] = 2 * np.arange(HALF) + 1
    return p


def _weight_perms():
    pr = _deinterleave_perm()
    qperm = np.arange(N_HEAD * HEAD_DIM, dtype=np.int32)
    for h in range(N_HEAD):
        base = h * HEAD_DIM + NOPE_DIM
        qperm[base:base + ROPE_DIM] = base + pr
    kvaperm = np.arange(KV_LORA + ROPE_DIM, dtype=np.int32)
    kvaperm[KV_LORA:] = KV_LORA + pr
    return qperm, kvaperm


def _k1_body(x_ref, ln1_ref, wq_ref, wkva_ref, wkvb_ref, cos_ref, sin_ref,
             q_ref, knope_ref, krope_ref):
    x = x_ref[...]
    xn = x * lax.rsqrt(jnp.mean(x * x, axis=1, keepdims=True) + 1e-6) * ln1_ref[...]
    xnb = xn.astype(jnp.bfloat16)
    q = jnp.dot(xnb, wq_ref[...].astype(jnp.bfloat16),
                preferred_element_type=jnp.float32)
    ckv = jnp.dot(xnb, wkva_ref[...].astype(jnp.bfloat16),
                  preferred_element_type=jnp.float32)
    latent = ckv[:, :KV_LORA]
    kr = ckv[:, KV_LORA:]
    knope_ref[...] = jnp.dot(latent.astype(jnp.bfloat16),
                             wkvb_ref[...].astype(jnp.bfloat16),
                             preferred_element_type=jnp.float32)
    cos = cos_ref[...]
    sin = sin_ref[...]
    q_ref[...] = q
    for h in range(N_HEAD):
        base = h * HEAD_DIM + NOPE_DIM
        a = q[:, base:base + HALF]
        b = q[:, base + HALF:base + ROPE_DIM]
        q_ref[:, base:base + HALF] = a * cos - b * sin
        q_ref[:, base + HALF:base + ROPE_DIM] = a * sin + b * cos
    a = kr[:, :HALF]
    b = kr[:, HALF:]
    krope_ref[...] = jnp.concatenate([a * cos - b * sin, a * sin + b * cos], axis=1)


def _proj(xf, ln1_w, Wq_p, Wkva_p, Wkvb, cosT, sinT):
    T = xf.shape[0]
    nb = T // BT1
    return pl.pallas_call(
        _k1_body,
        grid=(nb,),
        in_specs=[
            pl.BlockSpec((BT1, N_EMBD), lambda i: (i, 0)),
            pl.BlockSpec((1, N_EMBD), lambda i: (0, 0)),
            pl.BlockSpec((N_EMBD, N_HEAD * HEAD_DIM), lambda i: (0, 0)),
            pl.BlockSpec((N_EMBD, KV_LORA + ROPE_DIM), lambda i: (0, 0)),
            pl.BlockSpec((KV_LORA, N_HEAD * NOPE_DIM), lambda i: (0, 0)),
            pl.BlockSpec((BT1, HALF), lambda i: (i, 0)),
            pl.BlockSpec((BT1, HALF), lambda i: (i, 0)),
        ],
        out_specs=[
            pl.BlockSpec((BT1, N_HEAD * HEAD_DIM), lambda i: (i, 0)),
            pl.BlockSpec((BT1, N_HEAD * NOPE_DIM), lambda i: (i, 0)),
            pl.BlockSpec((BT1, ROPE_DIM), lambda i: (i, 0)),
        ],
        out_shape=[
            jax.ShapeDtypeStruct((T, N_HEAD * HEAD_DIM), jnp.float32),
            jax.ShapeDtypeStruct((T, N_HEAD * NOPE_DIM), jnp.float32),
            jax.ShapeDtypeStruct((T, ROPE_DIM), jnp.float32),
        ],
    )(xf, ln1_w, Wq_p, Wkva_p, Wkvb, cosT, sinT)


def _att_body_iq(iq, q_ref, kT_ref, k_ref, y_ref):
    L = (iq + 1) * BTQ
    q = q_ref[0].astype(jnp.bfloat16)
    kT = kT_ref[0].astype(jnp.bfloat16)
    v = k_ref[0].astype(jnp.bfloat16)
    s_diag = jnp.dot(q, kT[:, L - BTQ:],
                     preferred_element_type=jnp.float32) * 0.125
    row = lax.broadcasted_iota(jnp.int32, (BTQ, BTQ), 0)
    col = lax.broadcasted_iota(jnp.int32, (BTQ, BTQ), 1)
    s_diag = jnp.where(col <= row, s_diag, jnp.float32(-1e9))
    if iq == 0:
        m = jnp.max(s_diag, axis=1, keepdims=True)
        p = jnp.exp(s_diag - m).astype(jnp.bfloat16)
        l = jnp.sum(p.astype(jnp.float32), axis=1, keepdims=True)
        y = jnp.dot(p, v, preferred_element_type=jnp.float32)
    else:
        s_pre = jnp.dot(q, kT[:, :L - BTQ],
                        preferred_element_type=jnp.float32) * 0.125
        m = jnp.maximum(jnp.max(s_pre, axis=1, keepdims=True),
                        jnp.max(s_diag, axis=1, keepdims=True))
        p_pre = jnp.exp(s_pre - m).astype(jnp.bfloat16)
        p_diag = jnp.exp(s_diag - m).astype(jnp.bfloat16)
        l = (jnp.sum(p_pre.astype(jnp.float32), axis=1, keepdims=True)
             + jnp.sum(p_diag.astype(jnp.float32), axis=1, keepdims=True))
        y = (jnp.dot(p_pre, v[:L - BTQ], preferred_element_type=jnp.float32)
             + jnp.dot(p_diag, v[L - BTQ:], preferred_element_type=jnp.float32))
    y_ref[0] = y / l


def _attention(q3, kT3, k3):
    T = q3.shape[1]
    nq = T // BTQ
    pieces = []
    for iq in range(nq):
        L = (iq + 1) * BTQ
        pieces.append(pl.pallas_call(
            functools.partial(_att_body_iq, iq),
            grid=(N_HEAD,),
            in_specs=[
                pl.BlockSpec((1, BTQ, HEAD_DIM), lambda h, iq=iq: (h, iq, 0)),
                pl.BlockSpec((1, HEAD_DIM, L), lambda h: (h, 0, 0)),
                pl.BlockSpec((1, L, HEAD_DIM), lambda h: (h, 0, 0)),
            ],
            out_specs=pl.BlockSpec((1, BTQ, HEAD_DIM), lambda h: (h, 0, 0)),
            out_shape=jax.ShapeDtypeStruct((N_HEAD, BTQ, HEAD_DIM), jnp.float32),
        )(q3, kT3, k3))
    return jnp.concatenate(pieces, axis=1)


def _oproj_gate_body(x_ref, y_ref, wo_ref, ln2_ref, gw_ref, bias_ref,
                     h_ref, xn2_ref, ti_ref, tw_ref):
    h = x_ref[...] + jnp.dot(y_ref[...].astype(jnp.bfloat16),
                             wo_ref[...].astype(jnp.bfloat16),
                             preferred_element_type=jnp.float32)
    h_ref[...] = h
    xn = h * lax.rsqrt(jnp.mean(h * h, axis=1, keepdims=True) + 1e-6) * ln2_ref[...]
    xn2_ref[...] = xn
    logits = jnp.dot(xn, gw_ref[...], preferred_element_type=jnp.float32) + bias_ref[...]
    m = jnp.max(logits, axis=1, keepdims=True)
    e = jnp.exp(logits - m)
    probs = e / jnp.sum(e, axis=1, keepdims=True)
    lane = lax.broadcasted_iota(jnp.int32, probs.shape, 1)
    m1 = jnp.max(probs, axis=1, keepdims=True)
    sel1 = jnp.min(jnp.where(probs == m1, lane, 99), axis=1, keepdims=True)
    p2 = jnp.where(lane == sel1, jnp.float32(-1.0), probs)
    m2 = jnp.max(p2, axis=1, keepdims=True)
    sel2 = jnp.min(jnp.where(p2 == m2, lane, 99), axis=1, keepdims=True)
    denom = m1 + m2
    ti_ref[...] = jnp.where(lane == 0, sel1, jnp.where(lane == 1, sel2, 0))
    tw_ref[...] = jnp.where(lane == 0, m1 / denom,
                            jnp.where(lane == 1, m2 / denom, jnp.float32(0.0)))


def _oproj_gate(xf, y, Wo_p, ln2_w, gateW, bias):
    T = xf.shape[0]
    return pl.pallas_call(
        _oproj_gate_body,
        grid=(T // BT1,),
        in_specs=[
            pl.BlockSpec((BT1, N_EMBD), lambda i: (i, 0)),
            pl.BlockSpec((BT1, N_HEAD * HEAD_DIM), lambda i: (i, 0)),
            pl.BlockSpec((N_HEAD * HEAD_DIM, N_EMBD), lambda i: (0, 0)),
            pl.BlockSpec((1, N_EMBD), lambda i: (0, 0)),
            pl.BlockSpec((N_EMBD, N_EXP), lambda i: (0, 0)),
            pl.BlockSpec((1, N_EXP), lambda i: (0, 0)),
        ],
        out_specs=[
            pl.BlockSpec((BT1, N_EMBD), lambda i: (i, 0)),
            pl.BlockSpec((BT1, N_EMBD), lambda i: (i, 0)),
            pl.BlockSpec((BT1, N_EXP), lambda i: (i, 0)),
            pl.BlockSpec((BT1, N_EXP), lambda i: (i, 0)),
        ],
        out_shape=[
            jax.ShapeDtypeStruct((T, N_EMBD), jnp.float32),
            jax.ShapeDtypeStruct((T, N_EMBD), jnp.float32),
            jax.ShapeDtypeStruct((T, N_EXP), jnp.int32),
            jax.ShapeDtypeStruct((T, N_EXP), jnp.float32),
        ],
    )(xf, y, Wo_p, ln2_w, gateW, bias)


def _expert_body(be_ref, act_ref, xs_ref, sw_ref, gw_ref, uw_ref, dw_ref, ys_ref):
    i = pl.program_id(0)

    @pl.when(act_ref[i] != 0)
    def _():
        x = xs_ref[...].astype(jnp.bfloat16)
        g = jnp.dot(x, gw_ref[0].astype(jnp.bfloat16),
                    preferred_element_type=jnp.float32)
        u = jnp.dot(x, uw_ref[0].astype(jnp.bfloat16),
                    preferred_element_type=jnp.float32)
        a = (g * jax.nn.sigmoid(g) * u).astype(jnp.bfloat16)
        ys_ref[...] = jnp.dot(a, dw_ref[0].astype(jnp.bfloat16),
                              preferred_element_type=jnp.float32) * sw_ref[...]


def _experts(be, act, xs, swt, gw, uw, dw):
    NBT = xs.shape[0]
    NB = NBT // BTE
    grid_spec = pltpu.PrefetchScalarGridSpec(
        num_scalar_prefetch=2,
        grid=(NB,),
        in_specs=[
            pl.BlockSpec((BTE, N_EMBD), lambda i, be, act: (i, 0)),
            pl.BlockSpec((BTE, 1), lambda i, be, act: (i, 0)),
            pl.BlockSpec((1, N_EMBD, INTER), lambda i, be, act: (be[i], 0, 0)),
            pl.BlockSpec((1, N_EMBD, INTER), lambda i, be, act: (be[i], 0, 0)),
            pl.BlockSpec((1, INTER, N_EMBD), lambda i, be, act: (be[i], 0, 0)),
        ],
        out_specs=pl.BlockSpec((BTE, N_EMBD), lambda i, be, act: (i, 0)),
    )
    return pl.pallas_call(
        _expert_body,
        grid_spec=grid_spec,
        out_shape=jax.ShapeDtypeStruct((NBT, N_EMBD), jnp.float32),
    )(be, act, xs, swt, gw, uw, dw)


def _combine_body(h_ref, xn2_ref, y0_ref, y1_ref, sgw_ref, suw_ref, sdw_ref,
                  out_ref):
    x = xn2_ref[...].astype(jnp.bfloat16)
    g = jnp.dot(x, sgw_ref[...].astype(jnp.bfloat16),
                preferred_element_type=jnp.float32)
    u = jnp.dot(x, suw_ref[...].astype(jnp.bfloat16),
                preferred_element_type=jnp.float32)
    a = (g * jax.nn.sigmoid(g) * u).astype(jnp.bfloat16)
    shared = jnp.dot(a, sdw_ref[...].astype(jnp.bfloat16),
                     preferred_element_type=jnp.float32)
    out_ref[...] = h_ref[...] + shared + y0_ref[...] + y1_ref[...]


def _combine(h, xn2, yall, sgw, suw, sdw):
    T = h.shape[0]
    nb = T // BTS
    return pl.pallas_call(
        _combine_body,
        grid=(nb,),
        in_specs=[
            pl.BlockSpec((BTS, N_EMBD), lambda i: (i, 0)),
            pl.BlockSpec((BTS, N_EMBD), lambda i: (i, 0)),
            pl.BlockSpec((BTS, N_EMBD), lambda i: (i, 0)),
            pl.BlockSpec((BTS, N_EMBD), lambda i, nb=nb: (i + nb, 0)),
            pl.BlockSpec((N_EMBD, INTER), lambda i: (0, 0)),
            pl.BlockSpec((N_EMBD, INTER), lambda i: (0, 0)),
            pl.BlockSpec((INTER, N_EMBD), lambda i: (0, 0)),
        ],
        out_specs=pl.BlockSpec((BTS, N_EMBD), lambda i: (i, 0)),
        out_shape=jax.ShapeDtypeStruct((T, N_EMBD), jnp.float32),
    )(h, xn2, yall, yall, sgw, suw, sdw)


def _sc_scatter_stream(pos, pw, zeros_i, zeros_f):
    P2 = pos.shape[0]
    NBT = zeros_i.shape[0]
    mesh = plsc.VectorSubcoreMesh(core_axis_name="c", subcore_axis_name="s")

    @functools.partial(
        pl.kernel, mesh=mesh,
        out_type=[jax.ShapeDtypeStruct((NBT,), jnp.int32),
                  jax.ShapeDtypeStruct((NBT,), jnp.float32)],
        compiler_params=pltpu.CompilerParams(needs_layout_passes=False),
        scratch_types=[pltpu.VMEM((P2,), jnp.int32),
                       pltpu.VMEM((P2,), jnp.float32),
                       pltpu.VMEM((NBT,), jnp.int32),
                       pltpu.VMEM((NBT,), jnp.float32)],
    )
    def k(pos_hbm, pw_hbm, zi_hbm, zf_hbm, sti_hbm, swt_hbm,
          pos_v, pw_v, sti_v, swt_v):
        wid = lax.axis_index("s") * 2 + lax.axis_index("c")

        @pl.when(wid == 0)
        def _():
            pltpu.sync_copy(pos_hbm, pos_v)
            pltpu.sync_copy(pw_hbm, pw_v)
            pltpu.sync_copy(zi_hbm, sti_v)
            pltpu.sync_copy(zf_hbm, swt_v)

            @pl.loop(0, P2 // 16)
            def body(i):
                idx = pos_v[pl.ds(i * 16, 16)]
                toks = (i * 16
                        + lax.broadcasted_iota(jnp.int32, (16,), 0)) // TOP_K
                plsc.store_scatter(sti_v, [idx], toks)
                plsc.store_scatter(swt_v, [idx], pw_v[pl.ds(i * 16, 16)])

            pltpu.sync_copy(sti_v, sti_hbm)
            pltpu.sync_copy(swt_v, swt_hbm)

    return k(pos, pw, zeros_i, zeros_f)


def _sc_gather_rows(table, idx, chunk):
    V, D = table.shape
    B = idx.shape[0]
    NW = 32
    b_per_w = B // NW
    nch = b_per_w // chunk
    mesh = plsc.VectorSubcoreMesh(core_axis_name="c", subcore_axis_name="s")

    @functools.partial(
        pl.kernel, mesh=mesh,
        out_type=jax.ShapeDtypeStruct((B, D), jnp.float32),
        scratch_types=[pltpu.VMEM((chunk,), jnp.int32),
                       pltpu.VMEM((chunk, D), jnp.float32),
                       pltpu.SemaphoreType.DMA],
    )
    def k(table_hbm, idx_hbm, out_hbm, idx_v, rows_v, sem):
        wid = lax.axis_index("s") * 2 + lax.axis_index("c")
        base = wid * b_per_w
        for c in range(nch):
            off = base + c * chunk
            pltpu.sync_copy(idx_hbm.at[pl.ds(off, chunk)], idx_v)
            pltpu.async_copy(table_hbm.at[idx_v], rows_v, sem).wait()
            pltpu.sync_copy(rows_v, out_hbm.at[pl.ds(off, chunk)])

    return k(table, idx)


def _route_meta(ti, tw, T):
    P2 = T * TOP_K
    eid = ti[:, :TOP_K].reshape(-1)
    pw = tw[:, :TOP_K].reshape(-1)
    oh = (eid[:, None] == jnp.arange(N_EXP, dtype=jnp.int32)[None, :]).astype(jnp.int32)
    csum = jnp.cumsum(oh, axis=0)
    counts = csum[-1]
    rank_within = jnp.sum(oh * (csum - 1), axis=1)
    nblk = (counts + BTE - 1) // BTE
    ends = jnp.cumsum(nblk)
    bstart = ends - nblk
    pos = jnp.sum(oh * bstart[None, :], axis=1) * BTE + rank_within
    NB = P2 // BTE + N_EXP
    bids = jnp.arange(NB, dtype=jnp.int32)
    be = jnp.searchsorted(ends, bids, side='right').astype(jnp.int32)
    act = (bids < ends[-1]).astype(jnp.int32)
    be = jnp.where(act == 1, be, 0)
    return pos, pw, be, act


def kernel(x, ln1_w, ln2_w, Wq, Wkva, Wkvb, Wo, gateW, expert_bias, gw, uw, dw, sgw, suw, sdw):
    B, T, C = x.shape
    xf = x.reshape(T, C)

    cos_np, sin_np = _rope_tables(T)
    cosT = jnp.asarray(cos_np)
    sinT = jnp.asarray(sin_np)
    qperm, kvaperm = _weight_perms()
    Wq_p = jnp.take(Wq, qperm, axis=1)
    Wkva_p = jnp.take(Wkva, kvaperm, axis=1)
    Wo_p = jnp.take(Wo, qperm, axis=0)

    q, knope, krope = _proj(xf, ln1_w.reshape(1, C), Wq_p, Wkva_p, Wkvb, cosT, sinT)
    q3 = q.reshape(T, N_HEAD, HEAD_DIM).transpose(1, 0, 2)
    knope3 = knope.reshape(T, N_HEAD, NOPE_DIM).transpose(1, 0, 2)
    k3 = jnp.concatenate(
        [knope3, jnp.broadcast_to(krope[None], (N_HEAD, T, ROPE_DIM))], axis=2)
    kT3 = k3.transpose(0, 2, 1)
    y3 = _attention(q3, kT3, k3)
    y = y3.transpose(1, 0, 2).reshape(T, N_HEAD * HEAD_DIM)
    h, xn2, ti, tw = _oproj_gate(xf, y, Wo_p, ln2_w.reshape(1, C), gateW,
                                 expert_bias.reshape(1, N_EXP))
    pos, pw, be, act = _route_meta(ti, tw, T)
    NBT = (T * TOP_K // BTE + N_EXP) * BTE

    sti, swt = _sc_scatter_stream(pos, pw, jnp.zeros((NBT,), jnp.int32),
                                  jnp.zeros((NBT,), jnp.float32))
    xs = jnp.take(xn2, sti, axis=0)
    ysw = _experts(be, act, xs, swt[:, None], gw, uw, dw)
    posT = pos.reshape(T, TOP_K)
    pos2 = jnp.concatenate([posT[:, 0], posT[:, 1]])
    yall = jnp.take(ysw, pos2, axis=0)

    out = _combine(h, xn2, yall, sgw, suw, sdw)
    return out.reshape(B, T, C)
